# baseline (speedup 1.0000x reference)
import functools

import jax
import jax.numpy as jnp
from jax import lax
from jax.experimental import pallas as pl
from jax.experimental.pallas import tpu as pltpu
from jax.experimental.pallas import tpu_sc as plsc

_B = 32
_H = 512
_W = 512

_INFO = plsc.get_sparse_core_info()
_NC = _INFO.num_cores
_NS = _INFO.num_subcores
_L = _INFO.num_lanes
_NW = _NC * _NS

_R_SC = 256
_ROWS = 32
_NCHUNK = _R_SC // _ROWS
_VECS = _ROWS * _W // _L
_RVECS = _W // _L

_BR = 64
_R_TC = _H - _R_SC
_NBLK = _R_TC // _BR


def _make_sc_kernel():
    mesh = plsc.VectorSubcoreMesh(core_axis_name="c", subcore_axis_name="s")

    @functools.partial(
        pl.kernel,
        mesh=mesh,
        out_type=jax.ShapeDtypeStruct((_NW, 2, _L), jnp.float32),
        scratch_types=[
            pltpu.VMEM((_ROWS, _W), jnp.float32),
            pltpu.VMEM((_ROWS, _W), jnp.float32),
            pltpu.VMEM((_ROWS, _W), jnp.float32),
            pltpu.VMEM((_ROWS, _W), jnp.float32),
            pltpu.VMEM((_ROWS, _W), jnp.float32),
            pltpu.VMEM((_ROWS, _W), jnp.float32),
            pltpu.VMEM((2, _L), jnp.float32),
            pltpu.SemaphoreType.DMA,
            pltpu.SemaphoreType.DMA,
        ],
    )
    def masked_gauss(m_hbm, lv_hbm, t_hbm, out_hbm,
                     m0, m1, lv0, lv1, t0, t1, acc, sem0, sem1):
        wid = lax.axis_index("s") * _NC + lax.axis_index("c")
        bufs = ((m0, lv0, t0, sem0), (m1, lv1, t1, sem1))

        def start(g, slot):
            mb, lvb, tb, sem = bufs[slot]
            rows = pl.ds(g * _ROWS, _ROWS)
            return (
                pltpu.async_copy(m_hbm.at[wid, 0, rows], mb, sem),
                pltpu.async_copy(lv_hbm.at[wid, 0, rows], lvb, sem),
                pltpu.async_copy(t_hbm.at[wid, rows], tb, sem),
            )

        zero = jnp.zeros((_L,), jnp.float32)
        carry = (zero, zero)
        inflight = start(0, 0)
        for g in range(_NCHUNK):
            slot = g % 2
            nxt = start(g + 1, 1 - slot) if g + 1 < _NCHUNK else None
            for h in inflight:
                h.wait()
            mb, lvb, tb, _ = bufs[slot]

            def body(i, c, mb=mb, lvb=lvb, tb=tb):
                s, n = c
                r = lax.shift_right_logical(i, 5)
                col = pl.multiple_of(
                    lax.shift_left(lax.bitwise_and(i, _RVECS - 1), 4), _L)
                sl = pl.ds(col, _L)
                m = mb[r, sl]
                lv = lvb[r, sl]
                t = tb[r, sl]
                msk = t > 0.0
                d = t - m
                e = jnp.exp(-lv) * (d * d) + lv
                return (s + jnp.where(msk, e, 0.0),
                        n + jnp.where(msk, 1.0, 0.0))

            carry = lax.fori_loop(0, _VECS, body, carry, unroll=8)
            inflight = nxt
        s, n = carry
        acc[0, :] = s
        acc[1, :] = n
        pltpu.sync_copy(acc, out_hbm.at[wid])

    return masked_gauss


_sc_call = _make_sc_kernel()


def _tc_body(m_ref, lv_ref, t_ref, s_ref, c_ref):
    @pl.when((pl.program_id(0) == 0) & (pl.program_id(1) == 0))
    def _init():
        s_ref[...] = jnp.zeros_like(s_ref)
        c_ref[...] = jnp.zeros_like(c_ref)

    m = m_ref[0, 0]
    lv = lv_ref[0, 0]
    t = t_ref[0]
    msk = t > 0.0
    d = t - m
    e = jnp.exp(-lv) * (d * d) + lv
    xs = jnp.where(msk, e, 0.0).reshape(_BR // 8, 8, 4, 128)
    xc = jnp.where(msk, 1.0, 0.0).reshape(_BR // 8, 8, 4, 128)
    s_ref[...] += xs.sum(axis=(0, 2))
    c_ref[...] += xc.sum(axis=(0, 2))


_tc_call = pl.pallas_call(
    _tc_body,
    grid=(_B, _NBLK),
    in_specs=[
        pl.BlockSpec((1, 1, _BR, _W),
                     lambda b, n: (b, 0, n + _R_SC // _BR, 0)),
        pl.BlockSpec((1, 1, _BR, _W),
                     lambda b, n: (b, 0, n + _R_SC // _BR, 0)),
        pl.BlockSpec((1, _BR, _W),
                     lambda b, n: (b, n + _R_SC // _BR, 0)),
    ],
    out_specs=[
        pl.BlockSpec((8, 128), lambda b, n: (0, 0)),
        pl.BlockSpec((8, 128), lambda b, n: (0, 0)),
    ],
    out_shape=[
        jax.ShapeDtypeStruct((8, 128), jnp.float32),
        jax.ShapeDtypeStruct((8, 128), jnp.float32),
    ],
    compiler_params=pltpu.CompilerParams(
        dimension_semantics=("arbitrary", "arbitrary")),
)


@jax.jit
def kernel(means, log_vars, targets):
    sc_parts = _sc_call(means, log_vars, targets)
    tc_sum, tc_cnt = _tc_call(means, log_vars, targets)
    total = sc_parts[:, 0, :].sum() + tc_sum.sum()
    count = sc_parts[:, 1, :].sum() + tc_cnt.sum()
    return total / count


# speedup vs baseline: 1.4088x; 1.4088x over previous
import functools

import jax
import jax.numpy as jnp
from jax import lax
from jax.experimental import pallas as pl
from jax.experimental.pallas import tpu as pltpu
from jax.experimental.pallas import tpu_sc as plsc

_B = 32
_H = 512
_W = 512

_INFO = plsc.get_sparse_core_info()
_NC = _INFO.num_cores
_NS = _INFO.num_subcores
_L = _INFO.num_lanes
_NW = _NC * _NS

_R_SC = 256
_ROWS = 32
_NCHUNK = _R_SC // _ROWS
_VECS = _ROWS * _W // _L
_RVECS = _W // _L

_BR = 128
_R_TC = _H - _R_SC
_NBLK = _R_TC // _BR


def _make_sc_kernel():
    mesh = plsc.VectorSubcoreMesh(core_axis_name="c", subcore_axis_name="s")

    @functools.partial(
        pl.kernel,
        mesh=mesh,
        out_type=jax.ShapeDtypeStruct((_NW, 2, _L), jnp.float32),
        scratch_types=[
            pltpu.VMEM((_ROWS, _W), jnp.float32),
            pltpu.VMEM((_ROWS, _W), jnp.float32),
            pltpu.VMEM((_ROWS, _W), jnp.float32),
            pltpu.VMEM((_ROWS, _W), jnp.float32),
            pltpu.VMEM((_ROWS, _W), jnp.float32),
            pltpu.VMEM((_ROWS, _W), jnp.float32),
            pltpu.VMEM((2, _L), jnp.float32),
            pltpu.SemaphoreType.DMA,
            pltpu.SemaphoreType.DMA,
        ],
    )
    def masked_gauss(m_hbm, lv_hbm, t_hbm, out_hbm,
                     m0, m1, lv0, lv1, t0, t1, acc, sem0, sem1):
        wid = lax.axis_index("s") * _NC + lax.axis_index("c")
        bufs = ((m0, lv0, t0, sem0), (m1, lv1, t1, sem1))

        def start(g, slot):
            mb, lvb, tb, sem = bufs[slot]
            rows = pl.ds(g * _ROWS, _ROWS)
            return (
                pltpu.async_copy(m_hbm.at[wid, 0, rows], mb, sem),
                pltpu.async_copy(lv_hbm.at[wid, 0, rows], lvb, sem),
                pltpu.async_copy(t_hbm.at[wid, rows], tb, sem),
            )

        zero = jnp.zeros((_L,), jnp.float32)
        carry = (zero, zero)
        inflight = start(0, 0)
        for g in range(_NCHUNK):
            slot = g % 2
            nxt = start(g + 1, 1 - slot) if g + 1 < _NCHUNK else None
            for h in inflight:
                h.wait()
            mb, lvb, tb, _ = bufs[slot]

            def body(i, c, mb=mb, lvb=lvb, tb=tb):
                s, n = c
                r = lax.shift_right_logical(i, 5)
                col = pl.multiple_of(
                    lax.shift_left(lax.bitwise_and(i, _RVECS - 1), 4), _L)
                sl = pl.ds(col, _L)
                m = mb[r, sl]
                lv = lvb[r, sl]
                t = tb[r, sl]
                msk = t > 0.0
                d = t - m
                e = jnp.exp(-lv) * (d * d) + lv
                return (s + jnp.where(msk, e, 0.0),
                        n + jnp.where(msk, 1.0, 0.0))

            carry = lax.fori_loop(0, _VECS, body, carry, unroll=8)
            inflight = nxt
        s, n = carry
        acc[0, :] = s
        acc[1, :] = n
        pltpu.sync_copy(acc, out_hbm.at[wid])

    return masked_gauss


_sc_call = _make_sc_kernel()


def _tc_body(m_ref, lv_ref, t_ref, s_ref, c_ref):
    @pl.when((pl.program_id(0) == 0) & (pl.program_id(1) == 0))
    def _init():
        s_ref[...] = jnp.zeros_like(s_ref)
        c_ref[...] = jnp.zeros_like(c_ref)

    m = m_ref[0, 0]
    lv = lv_ref[0, 0]
    t = t_ref[0]
    msk = t > 0.0
    d = t - m
    e = jnp.exp(-lv) * (d * d) + lv
    xs = jnp.where(msk, e, 0.0).reshape(_BR // 8, 8, _W)
    xc = jnp.where(msk, 1.0, 0.0).reshape(_BR // 8, 8, _W)
    s_ref[...] += xs.sum(axis=0)
    c_ref[...] += xc.sum(axis=0)


_tc_call = pl.pallas_call(
    _tc_body,
    grid=(_B, _NBLK),
    in_specs=[
        pl.BlockSpec((1, 1, _BR, _W),
                     lambda b, n: (b, 0, n + _R_SC // _BR, 0)),
        pl.BlockSpec((1, 1, _BR, _W),
                     lambda b, n: (b, 0, n + _R_SC // _BR, 0)),
        pl.BlockSpec((1, _BR, _W),
                     lambda b, n: (b, n + _R_SC // _BR, 0)),
    ],
    out_specs=[
        pl.BlockSpec((8, _W), lambda b, n: (0, 0)),
        pl.BlockSpec((8, _W), lambda b, n: (0, 0)),
    ],
    out_shape=[
        jax.ShapeDtypeStruct((8, _W), jnp.float32),
        jax.ShapeDtypeStruct((8, _W), jnp.float32),
    ],
    compiler_params=pltpu.CompilerParams(
        dimension_semantics=("arbitrary", "arbitrary")),
)


@jax.jit
def kernel(means, log_vars, targets):
    sc_parts = _sc_call(means, log_vars, targets)
    tc_sum, tc_cnt = _tc_call(means, log_vars, targets)
    total = sc_parts[:, 0, :].sum() + tc_sum.sum()
    count = sc_parts[:, 1, :].sum() + tc_cnt.sum()
    return total / count


# speedup vs baseline: 1.6884x; 1.1985x over previous
import functools

import jax
import jax.numpy as jnp
from jax import lax
from jax.experimental import pallas as pl
from jax.experimental.pallas import tpu as pltpu
from jax.experimental.pallas import tpu_sc as plsc

_B = 32
_H = 512
_W = 512

_INFO = plsc.get_sparse_core_info()
_NC = _INFO.num_cores
_NS = _INFO.num_subcores
_L = _INFO.num_lanes
_NW = _NC * _NS

_R_SC = 256
_ROWS = 32
_NCHUNK = _R_SC // _ROWS
_VECS = _ROWS * _W // _L
_RVECS = _W // _L

_BR = 256
_R_TC = _H - _R_SC
_NBLK = _R_TC // _BR


def _make_sc_kernel():
    mesh = plsc.VectorSubcoreMesh(core_axis_name="c", subcore_axis_name="s")

    @functools.partial(
        pl.kernel,
        mesh=mesh,
        out_type=jax.ShapeDtypeStruct((_NW, 2, _L), jnp.float32),
        scratch_types=[
            pltpu.VMEM((_ROWS, _W), jnp.float32),
            pltpu.VMEM((_ROWS, _W), jnp.float32),
            pltpu.VMEM((_ROWS, _W), jnp.float32),
            pltpu.VMEM((_ROWS, _W), jnp.float32),
            pltpu.VMEM((_ROWS, _W), jnp.float32),
            pltpu.VMEM((_ROWS, _W), jnp.float32),
            pltpu.VMEM((2, _L), jnp.float32),
            pltpu.SemaphoreType.DMA,
            pltpu.SemaphoreType.DMA,
        ],
    )
    def masked_gauss(m_hbm, lv_hbm, t_hbm, out_hbm,
                     m0, m1, lv0, lv1, t0, t1, acc, sem0, sem1):
        wid = lax.axis_index("s") * _NC + lax.axis_index("c")
        bufs = ((m0, lv0, t0, sem0), (m1, lv1, t1, sem1))

        def start(g, slot):
            mb, lvb, tb, sem = bufs[slot]
            rows = pl.ds(g * _ROWS, _ROWS)
            return (
                pltpu.async_copy(m_hbm.at[wid, 0, rows], mb, sem),
                pltpu.async_copy(lv_hbm.at[wid, 0, rows], lvb, sem),
                pltpu.async_copy(t_hbm.at[wid, rows], tb, sem),
            )

        zero = jnp.zeros((_L,), jnp.float32)
        carry = (zero, zero)
        inflight = start(0, 0)
        for g in range(_NCHUNK):
            slot = g % 2
            nxt = start(g + 1, 1 - slot) if g + 1 < _NCHUNK else None
            for h in inflight:
                h.wait()
            mb, lvb, tb, _ = bufs[slot]

            def body(i, c, mb=mb, lvb=lvb, tb=tb):
                s, n = c
                r = lax.shift_right_logical(i, 5)
                col = pl.multiple_of(
                    lax.shift_left(lax.bitwise_and(i, _RVECS - 1), 4), _L)
                sl = pl.ds(col, _L)
                m = mb[r, sl]
                lv = lvb[r, sl]
                t = tb[r, sl]
                msk = t > 0.0
                d = t - m
                e = jnp.exp(-lv) * (d * d) + lv
                return (s + jnp.where(msk, e, 0.0),
                        n + jnp.where(msk, 1.0, 0.0))

            carry = lax.fori_loop(0, _VECS, body, carry, unroll=8)
            inflight = nxt
        s, n = carry
        acc[0, :] = s
        acc[1, :] = n
        pltpu.sync_copy(acc, out_hbm.at[wid])

    return masked_gauss


_sc_call = _make_sc_kernel()


def _tc_body(m_ref, lv_ref, t_ref, s_ref, c_ref):
    @pl.when((pl.program_id(0) == 0) & (pl.program_id(1) == 0))
    def _init():
        s_ref[...] = jnp.zeros_like(s_ref)
        c_ref[...] = jnp.zeros_like(c_ref)

    m = m_ref[0, 0]
    lv = lv_ref[0, 0]
    t = t_ref[0]
    msk = t > 0.0
    d = t - m
    e = jnp.exp(-lv) * (d * d) + lv
    xs = jnp.where(msk, e, 0.0).reshape(_BR // 8, 8, _W)
    xc = jnp.where(msk, 1.0, 0.0).reshape(_BR // 8, 8, _W)
    s_ref[...] += xs.sum(axis=0)
    c_ref[...] += xc.sum(axis=0)


_tc_call = pl.pallas_call(
    _tc_body,
    grid=(_B, _NBLK),
    in_specs=[
        pl.BlockSpec((1, 1, _BR, _W),
                     lambda b, n: (b, 0, n + _R_SC // _BR, 0)),
        pl.BlockSpec((1, 1, _BR, _W),
                     lambda b, n: (b, 0, n + _R_SC // _BR, 0)),
        pl.BlockSpec((1, _BR, _W),
                     lambda b, n: (b, n + _R_SC // _BR, 0)),
    ],
    out_specs=[
        pl.BlockSpec((8, _W), lambda b, n: (0, 0)),
        pl.BlockSpec((8, _W), lambda b, n: (0, 0)),
    ],
    out_shape=[
        jax.ShapeDtypeStruct((8, _W), jnp.float32),
        jax.ShapeDtypeStruct((8, _W), jnp.float32),
    ],
    compiler_params=pltpu.CompilerParams(
        dimension_semantics=("arbitrary", "arbitrary")),
)


@jax.jit
def kernel(means, log_vars, targets):
    sc_parts = _sc_call(means, log_vars, targets)
    tc_sum, tc_cnt = _tc_call(means, log_vars, targets)
    total = sc_parts[:, 0, :].sum() + tc_sum.sum()
    count = sc_parts[:, 1, :].sum() + tc_cnt.sum()
    return total / count


# speedup vs baseline: 1.7866x; 1.0581x over previous
import functools

import jax
import jax.numpy as jnp
from jax import lax
from jax.experimental import pallas as pl
from jax.experimental.pallas import tpu as pltpu
from jax.experimental.pallas import tpu_sc as plsc

_B = 32
_H = 512
_W = 512

_INFO = plsc.get_sparse_core_info()
_NC = _INFO.num_cores
_NS = _INFO.num_subcores
_L = _INFO.num_lanes
_NW = _NC * _NS

_R_SC = 256
_ROWS = 32
_NCHUNK = _R_SC // _ROWS
_VECS = _ROWS * _W // _L
_RVECS = _W // _L

_BR = 256
_R_TC = _H - _R_SC
_NBLK = _R_TC // _BR


def _make_sc_kernel():
    mesh = plsc.VectorSubcoreMesh(core_axis_name="c", subcore_axis_name="s")

    @functools.partial(
        pl.kernel,
        mesh=mesh,
        out_type=jax.ShapeDtypeStruct((_NW, 2, _L), jnp.float32),
        scratch_types=[
            pltpu.VMEM((_ROWS, _W), jnp.float32),
            pltpu.VMEM((_ROWS, _W), jnp.float32),
            pltpu.VMEM((_ROWS, _W), jnp.float32),
            pltpu.VMEM((_ROWS, _W), jnp.float32),
            pltpu.VMEM((_ROWS, _W), jnp.float32),
            pltpu.VMEM((_ROWS, _W), jnp.float32),
            pltpu.VMEM((2, _L), jnp.float32),
            pltpu.SemaphoreType.DMA,
            pltpu.SemaphoreType.DMA,
        ],
    )
    def masked_gauss(m_hbm, lv_hbm, t_hbm, out_hbm,
                     m0, m1, lv0, lv1, t0, t1, acc, sem0, sem1):
        wid = lax.axis_index("s") * _NC + lax.axis_index("c")
        bufs = ((m0, lv0, t0, sem0), (m1, lv1, t1, sem1))

        def start(g, slot):
            mb, lvb, tb, sem = bufs[slot]
            rows = pl.ds(g * _ROWS, _ROWS)
            return (
                pltpu.async_copy(m_hbm.at[wid, 0, rows], mb, sem),
                pltpu.async_copy(lv_hbm.at[wid, 0, rows], lvb, sem),
                pltpu.async_copy(t_hbm.at[wid, rows], tb, sem),
            )

        zero = jnp.zeros((_L,), jnp.float32)
        carry = (zero, zero)
        inflight = start(0, 0)
        for g in range(_NCHUNK):
            slot = g % 2
            nxt = start(g + 1, 1 - slot) if g + 1 < _NCHUNK else None
            for h in inflight:
                h.wait()
            mb, lvb, tb, _ = bufs[slot]

            def body(i, c, mb=mb, lvb=lvb, tb=tb):
                s, n = c
                r = lax.shift_right_logical(i, 5)
                col = pl.multiple_of(
                    lax.shift_left(lax.bitwise_and(i, _RVECS - 1), 4), _L)
                sl = pl.ds(col, _L)
                m = mb[r, sl]
                lv = lvb[r, sl]
                t = tb[r, sl]
                msk = t > 0.0
                d = t - m
                e = jnp.exp(-lv) * (d * d) + lv
                return (s + jnp.where(msk, e, 0.0),
                        n + jnp.where(msk, 1.0, 0.0))

            carry = lax.fori_loop(0, _VECS, body, carry, unroll=8)
            inflight = nxt
        s, n = carry
        acc[0, :] = s
        acc[1, :] = n
        pltpu.sync_copy(acc, out_hbm.at[wid])

    return masked_gauss


_sc_call = _make_sc_kernel()


_RING = 3
_RG = 32
_TC_BLOCKS = [(b, r0)
              for b in range(_B)
              for r0 in range(_R_SC, _H, _BR)]


def _tc_body(m_hbm, lv_hbm, t_hbm, s_ref, c_ref, mb, lvb, tb, sems):
    nb = len(_TC_BLOCKS)

    def start(g):
        b, r0 = _TC_BLOCKS[g]
        slot = g % _RING
        rows = pl.ds(r0, _BR)
        return (
            pltpu.async_copy(m_hbm.at[b, 0, rows], mb.at[slot], sems.at[slot]),
            pltpu.async_copy(lv_hbm.at[b, 0, rows], lvb.at[slot],
                             sems.at[slot]),
            pltpu.async_copy(t_hbm.at[b, rows], tb.at[slot], sems.at[slot]),
        )

    handles = [None] * nb
    handles[0] = start(0)
    handles[1] = start(1)
    s = jnp.zeros((8, _W), jnp.float32)
    c = jnp.zeros((8, _W), jnp.float32)
    for g in range(nb):
        if g + 2 < nb:
            handles[g + 2] = start(g + 2)
        for h in handles[g]:
            h.wait()
        handles[g] = None
        slot = g % _RING
        for r in range(0, _BR, _RG):
            rows = pl.ds(r, _RG)
            m = mb[slot, rows, :]
            lv = lvb[slot, rows, :]
            t = tb[slot, rows, :]
            msk = t > 0.0
            d = t - m
            e = jnp.exp(-lv) * (d * d) + lv
            s = s + jnp.where(msk, e, 0.0).reshape(_RG // 8, 8, _W).sum(0)
            c = c + jnp.where(msk, 1.0, 0.0).reshape(_RG // 8, 8, _W).sum(0)
    s_ref[...] = s
    c_ref[...] = c


_tc_call = pl.pallas_call(
    _tc_body,
    in_specs=[
        pl.BlockSpec(memory_space=pl.ANY),
        pl.BlockSpec(memory_space=pl.ANY),
        pl.BlockSpec(memory_space=pl.ANY),
    ],
    out_specs=[
        pl.BlockSpec(memory_space=pltpu.MemorySpace.VMEM),
        pl.BlockSpec(memory_space=pltpu.MemorySpace.VMEM),
    ],
    out_shape=[
        jax.ShapeDtypeStruct((8, _W), jnp.float32),
        jax.ShapeDtypeStruct((8, _W), jnp.float32),
    ],
    scratch_shapes=[
        pltpu.VMEM((_RING, _BR, _W), jnp.float32),
        pltpu.VMEM((_RING, _BR, _W), jnp.float32),
        pltpu.VMEM((_RING, _BR, _W), jnp.float32),
        pltpu.SemaphoreType.DMA((_RING,)),
    ],
)


@jax.jit
def kernel(means, log_vars, targets):
    sc_parts = _sc_call(means, log_vars, targets)
    tc_sum, tc_cnt = _tc_call(means, log_vars, targets)
    total = sc_parts[:, 0, :].sum() + tc_sum.sum()
    count = sc_parts[:, 1, :].sum() + tc_cnt.sum()
    return total / count


# speedup vs baseline: 1.7883x; 1.0009x over previous
import functools

import jax
import jax.numpy as jnp
from jax import lax
from jax.experimental import pallas as pl
from jax.experimental.pallas import tpu as pltpu
from jax.experimental.pallas import tpu_sc as plsc

_B = 32
_H = 512
_W = 512

_INFO = plsc.get_sparse_core_info()
_NC = _INFO.num_cores
_NS = _INFO.num_subcores
_L = _INFO.num_lanes
_NW = _NC * _NS

_R_SC = 256
_ROWS = 32
_NCHUNK = _R_SC // _ROWS
_VECS = _ROWS * _W // _L
_RVECS = _W // _L

_BR = 256
_R_TC = _H - _R_SC
_NBLK = _R_TC // _BR


def _make_sc_kernel():
    mesh = plsc.VectorSubcoreMesh(core_axis_name="c", subcore_axis_name="s")

    @functools.partial(
        pl.kernel,
        mesh=mesh,
        out_type=jax.ShapeDtypeStruct((_NW, 2, _L), jnp.float32),
        scratch_types=[
            pltpu.VMEM((_ROWS, _W), jnp.float32),
            pltpu.VMEM((_ROWS, _W), jnp.float32),
            pltpu.VMEM((_ROWS, _W), jnp.float32),
            pltpu.VMEM((_ROWS, _W), jnp.float32),
            pltpu.VMEM((_ROWS, _W), jnp.float32),
            pltpu.VMEM((_ROWS, _W), jnp.float32),
            pltpu.VMEM((2, _L), jnp.float32),
            pltpu.SemaphoreType.DMA,
            pltpu.SemaphoreType.DMA,
        ],
    )
    def masked_gauss(m_hbm, lv_hbm, t_hbm, out_hbm,
                     m0, m1, lv0, lv1, t0, t1, acc, sem0, sem1):
        wid = lax.axis_index("s") * _NC + lax.axis_index("c")
        bufs = ((m0, lv0, t0, sem0), (m1, lv1, t1, sem1))

        def start(g, slot):
            mb, lvb, tb, sem = bufs[slot]
            rows = pl.ds(g * _ROWS, _ROWS)
            return (
                pltpu.async_copy(m_hbm.at[wid, 0, rows], mb, sem),
                pltpu.async_copy(lv_hbm.at[wid, 0, rows], lvb, sem),
                pltpu.async_copy(t_hbm.at[wid, rows], tb, sem),
            )

        def wait(g, slot):
            for h in start_descr(g, slot):
                h.wait()

        def start_descr(g, slot):
            mb, lvb, tb, sem = bufs[slot]
            rows = pl.ds(g * _ROWS, _ROWS)
            return (
                pltpu.make_async_copy(m_hbm.at[wid, 0, rows], mb, sem),
                pltpu.make_async_copy(lv_hbm.at[wid, 0, rows], lvb, sem),
                pltpu.make_async_copy(t_hbm.at[wid, rows], tb, sem),
            )

        def compute(slot, carry):
            mb, lvb, tb, _ = bufs[slot]

            def body(i, c):
                s, n = c
                r = lax.shift_right_logical(i, 5)
                col = pl.multiple_of(
                    lax.shift_left(lax.bitwise_and(i, _RVECS - 1), 4), _L)
                sl = pl.ds(col, _L)
                m = mb[r, sl]
                lv = lvb[r, sl]
                t = tb[r, sl]
                msk = t > 0.0
                d = t - m
                e = jnp.exp(-lv) * (d * d) + lv
                return (s + jnp.where(msk, e, 0.0),
                        n + jnp.where(msk, 1.0, 0.0))

            return lax.fori_loop(0, _VECS, body, carry, unroll=4)

        zero = jnp.zeros((_L,), jnp.float32)
        start(0, 0)

        def pair_body(j, carry):
            c0 = 2 * j
            start(c0 + 1, 1)
            wait(c0, 0)
            carry = compute(0, carry)

            @pl.when(c0 + 2 < _NCHUNK)
            def _():
                start(c0 + 2, 0)

            wait(c0 + 1, 1)
            return compute(1, carry)

        s, n = lax.fori_loop(0, _NCHUNK // 2, pair_body, (zero, zero))
        acc[0, :] = s
        acc[1, :] = n
        pltpu.sync_copy(acc, out_hbm.at[wid])

    return masked_gauss


_sc_call = _make_sc_kernel()


_RING = 3
_RG = 32
_TC_BLOCKS = [(b, r0)
              for b in range(_B)
              for r0 in range(_R_SC, _H, _BR)]


def _tc_body(m_hbm, lv_hbm, t_hbm, s_ref, c_ref, mb, lvb, tb, sems):
    nb = len(_TC_BLOCKS)

    def start(g):
        b, r0 = _TC_BLOCKS[g]
        slot = g % _RING
        rows = pl.ds(r0, _BR)
        return (
            pltpu.async_copy(m_hbm.at[b, 0, rows], mb.at[slot], sems.at[slot]),
            pltpu.async_copy(lv_hbm.at[b, 0, rows], lvb.at[slot],
                             sems.at[slot]),
            pltpu.async_copy(t_hbm.at[b, rows], tb.at[slot], sems.at[slot]),
        )

    handles = [None] * nb
    handles[0] = start(0)
    handles[1] = start(1)
    s = jnp.zeros((8, _W), jnp.float32)
    c = jnp.zeros((8, _W), jnp.float32)
    for g in range(nb):
        if g + 2 < nb:
            handles[g + 2] = start(g + 2)
        for h in handles[g]:
            h.wait()
        handles[g] = None
        slot = g % _RING
        for r in range(0, _BR, _RG):
            rows = pl.ds(r, _RG)
            m = mb[slot, rows, :]
            lv = lvb[slot, rows, :]
            t = tb[slot, rows, :]
            msk = t > 0.0
            d = t - m
            e = jnp.exp(-lv) * (d * d) + lv
            s = s + jnp.where(msk, e, 0.0).reshape(_RG // 8, 8, _W).sum(0)
            c = c + jnp.where(msk, 1.0, 0.0).reshape(_RG // 8, 8, _W).sum(0)
    s_ref[...] = s
    c_ref[...] = c


_tc_call = pl.pallas_call(
    _tc_body,
    in_specs=[
        pl.BlockSpec(memory_space=pl.ANY),
        pl.BlockSpec(memory_space=pl.ANY),
        pl.BlockSpec(memory_space=pl.ANY),
    ],
    out_specs=[
        pl.BlockSpec(memory_space=pltpu.MemorySpace.VMEM),
        pl.BlockSpec(memory_space=pltpu.MemorySpace.VMEM),
    ],
    out_shape=[
        jax.ShapeDtypeStruct((8, _W), jnp.float32),
        jax.ShapeDtypeStruct((8, _W), jnp.float32),
    ],
    scratch_shapes=[
        pltpu.VMEM((_RING, _BR, _W), jnp.float32),
        pltpu.VMEM((_RING, _BR, _W), jnp.float32),
        pltpu.VMEM((_RING, _BR, _W), jnp.float32),
        pltpu.SemaphoreType.DMA((_RING,)),
    ],
)


@jax.jit
def kernel(means, log_vars, targets):
    sc_parts = _sc_call(means, log_vars, targets)
    tc_sum, tc_cnt = _tc_call(means, log_vars, targets)
    total = sc_parts[:, 0, :].sum() + tc_sum.sum()
    count = sc_parts[:, 1, :].sum() + tc_cnt.sum()
    return total / count


# speedup vs baseline: 2.7997x; 1.5656x over previous
import functools

import jax
import jax.numpy as jnp
from jax import lax
from jax.experimental import pallas as pl
from jax.experimental.pallas import tpu as pltpu
from jax.experimental.pallas import tpu_sc as plsc

_B = 32
_H = 512
_W = 512

_INFO = plsc.get_sparse_core_info()
_NC = _INFO.num_cores
_NS = _INFO.num_subcores
_L = _INFO.num_lanes
_NW = _NC * _NS

_R_SC = 0
_ROWS = 32
_NCHUNK = _R_SC // _ROWS
_VECS = _ROWS * _W // _L
_RVECS = _W // _L

_BR = 256
_R_TC = _H - _R_SC
_NBLK = _R_TC // _BR


def _make_sc_kernel():
    mesh = plsc.VectorSubcoreMesh(core_axis_name="c", subcore_axis_name="s")

    @functools.partial(
        pl.kernel,
        mesh=mesh,
        out_type=jax.ShapeDtypeStruct((_NW, 2, _L), jnp.float32),
        scratch_types=[
            pltpu.VMEM((_ROWS, _W), jnp.float32),
            pltpu.VMEM((_ROWS, _W), jnp.float32),
            pltpu.VMEM((_ROWS, _W), jnp.float32),
            pltpu.VMEM((_ROWS, _W), jnp.float32),
            pltpu.VMEM((_ROWS, _W), jnp.float32),
            pltpu.VMEM((_ROWS, _W), jnp.float32),
            pltpu.VMEM((2, _L), jnp.float32),
            pltpu.SemaphoreType.DMA,
            pltpu.SemaphoreType.DMA,
        ],
    )
    def masked_gauss(m_hbm, lv_hbm, t_hbm, out_hbm,
                     m0, m1, lv0, lv1, t0, t1, acc, sem0, sem1):
        wid = lax.axis_index("s") * _NC + lax.axis_index("c")
        bufs = ((m0, lv0, t0, sem0), (m1, lv1, t1, sem1))

        def start(g, slot):
            mb, lvb, tb, sem = bufs[slot]
            rows = pl.ds(g * _ROWS, _ROWS)
            return (
                pltpu.async_copy(m_hbm.at[wid, 0, rows], mb, sem),
                pltpu.async_copy(lv_hbm.at[wid, 0, rows], lvb, sem),
                pltpu.async_copy(t_hbm.at[wid, rows], tb, sem),
            )

        def wait(g, slot):
            for h in start_descr(g, slot):
                h.wait()

        def start_descr(g, slot):
            mb, lvb, tb, sem = bufs[slot]
            rows = pl.ds(g * _ROWS, _ROWS)
            return (
                pltpu.make_async_copy(m_hbm.at[wid, 0, rows], mb, sem),
                pltpu.make_async_copy(lv_hbm.at[wid, 0, rows], lvb, sem),
                pltpu.make_async_copy(t_hbm.at[wid, rows], tb, sem),
            )

        def compute(slot, carry):
            mb, lvb, tb, _ = bufs[slot]

            def body(i, c):
                s, n = c
                r = lax.shift_right_logical(i, 5)
                col = pl.multiple_of(
                    lax.shift_left(lax.bitwise_and(i, _RVECS - 1), 4), _L)
                sl = pl.ds(col, _L)
                m = mb[r, sl]
                lv = lvb[r, sl]
                t = tb[r, sl]
                msk = t > 0.0
                d = t - m
                e = jnp.exp(-lv) * (d * d) + lv
                return (s + jnp.where(msk, e, 0.0),
                        n + jnp.where(msk, 1.0, 0.0))

            return lax.fori_loop(0, _VECS, body, carry, unroll=4)

        zero = jnp.zeros((_L,), jnp.float32)
        start(0, 0)

        def pair_body(j, carry):
            c0 = 2 * j
            start(c0 + 1, 1)
            wait(c0, 0)
            carry = compute(0, carry)

            @pl.when(c0 + 2 < _NCHUNK)
            def _():
                start(c0 + 2, 0)

            wait(c0 + 1, 1)
            return compute(1, carry)

        s, n = lax.fori_loop(0, _NCHUNK // 2, pair_body, (zero, zero))
        acc[0, :] = s
        acc[1, :] = n
        pltpu.sync_copy(acc, out_hbm.at[wid])

    return masked_gauss


_sc_call = _make_sc_kernel()


_RING = 3
_RG = 32
_TC_BLOCKS = [(b, r0)
              for b in range(_B)
              for r0 in range(_R_SC, _H, _BR)]


def _tc_body(m_hbm, lv_hbm, t_hbm, out_ref, mb, lvb, tb, sems):
    nb = len(_TC_BLOCKS)

    def start(g):
        b, r0 = _TC_BLOCKS[g]
        slot = g % _RING
        rows = pl.ds(r0, _BR)
        return (
            pltpu.async_copy(m_hbm.at[b, 0, rows], mb.at[slot], sems.at[slot]),
            pltpu.async_copy(lv_hbm.at[b, 0, rows], lvb.at[slot],
                             sems.at[slot]),
            pltpu.async_copy(t_hbm.at[b, rows], tb.at[slot], sems.at[slot]),
        )

    handles = [None] * nb
    handles[0] = start(0)
    handles[1] = start(1)
    s = jnp.zeros((8, _W), jnp.float32)
    c = jnp.zeros((8, _W), jnp.float32)
    for g in range(nb):
        if g + 2 < nb:
            handles[g + 2] = start(g + 2)
        for h in handles[g]:
            h.wait()
        handles[g] = None
        slot = g % _RING
        for r in range(0, _BR, _RG):
            rows = pl.ds(r, _RG)
            m = mb[slot, rows, :]
            lv = lvb[slot, rows, :]
            t = tb[slot, rows, :]
            msk = t > 0.0
            d = t - m
            e = jnp.exp(-lv) * (d * d) + lv
            s = s + jnp.where(msk, e, 0.0).reshape(_RG // 8, 8, _W).sum(0)
            c = c + jnp.where(msk, 1.0, 0.0).reshape(_RG // 8, 8, _W).sum(0)
    out_ref[0, 0] = jnp.sum(s) / jnp.sum(c)


_tc_call = pl.pallas_call(
    _tc_body,
    in_specs=[
        pl.BlockSpec(memory_space=pl.ANY),
        pl.BlockSpec(memory_space=pl.ANY),
        pl.BlockSpec(memory_space=pl.ANY),
    ],
    out_specs=pl.BlockSpec(memory_space=pltpu.MemorySpace.SMEM),
    out_shape=jax.ShapeDtypeStruct((1, 1), jnp.float32),
    scratch_shapes=[
        pltpu.VMEM((_RING, _BR, _W), jnp.float32),
        pltpu.VMEM((_RING, _BR, _W), jnp.float32),
        pltpu.VMEM((_RING, _BR, _W), jnp.float32),
        pltpu.SemaphoreType.DMA((_RING,)),
    ],
)


@jax.jit
def kernel(means, log_vars, targets):
    loss = _tc_call(means, log_vars, targets)
    return loss[0, 0]


# speedup vs baseline: 3.3295x; 1.1893x over previous
import functools

import jax
import jax.numpy as jnp
from jax import lax
from jax.experimental import pallas as pl
from jax.experimental.pallas import tpu as pltpu
from jax.experimental.pallas import tpu_sc as plsc

_B = 32
_H = 512
_W = 512

_INFO = plsc.get_sparse_core_info()
_NC = _INFO.num_cores
_NS = _INFO.num_subcores
_L = _INFO.num_lanes
_NW = _NC * _NS

_R_SC = 0
_ROWS = 32
_NCHUNK = _R_SC // _ROWS
_VECS = _ROWS * _W // _L
_RVECS = _W // _L

_BR = 256
_R_TC = _H - _R_SC
_NBLK = _R_TC // _BR


def _make_sc_kernel():
    mesh = plsc.VectorSubcoreMesh(core_axis_name="c", subcore_axis_name="s")

    @functools.partial(
        pl.kernel,
        mesh=mesh,
        out_type=jax.ShapeDtypeStruct((_NW, 2, _L), jnp.float32),
        scratch_types=[
            pltpu.VMEM((_ROWS, _W), jnp.float32),
            pltpu.VMEM((_ROWS, _W), jnp.float32),
            pltpu.VMEM((_ROWS, _W), jnp.float32),
            pltpu.VMEM((_ROWS, _W), jnp.float32),
            pltpu.VMEM((_ROWS, _W), jnp.float32),
            pltpu.VMEM((_ROWS, _W), jnp.float32),
            pltpu.VMEM((2, _L), jnp.float32),
            pltpu.SemaphoreType.DMA,
            pltpu.SemaphoreType.DMA,
        ],
    )
    def masked_gauss(m_hbm, lv_hbm, t_hbm, out_hbm,
                     m0, m1, lv0, lv1, t0, t1, acc, sem0, sem1):
        wid = lax.axis_index("s") * _NC + lax.axis_index("c")
        bufs = ((m0, lv0, t0, sem0), (m1, lv1, t1, sem1))

        def start(g, slot):
            mb, lvb, tb, sem = bufs[slot]
            rows = pl.ds(g * _ROWS, _ROWS)
            return (
                pltpu.async_copy(m_hbm.at[wid, 0, rows], mb, sem),
                pltpu.async_copy(lv_hbm.at[wid, 0, rows], lvb, sem),
                pltpu.async_copy(t_hbm.at[wid, rows], tb, sem),
            )

        def wait(g, slot):
            for h in start_descr(g, slot):
                h.wait()

        def start_descr(g, slot):
            mb, lvb, tb, sem = bufs[slot]
            rows = pl.ds(g * _ROWS, _ROWS)
            return (
                pltpu.make_async_copy(m_hbm.at[wid, 0, rows], mb, sem),
                pltpu.make_async_copy(lv_hbm.at[wid, 0, rows], lvb, sem),
                pltpu.make_async_copy(t_hbm.at[wid, rows], tb, sem),
            )

        def compute(slot, carry):
            mb, lvb, tb, _ = bufs[slot]

            def body(i, c):
                s, n = c
                r = lax.shift_right_logical(i, 5)
                col = pl.multiple_of(
                    lax.shift_left(lax.bitwise_and(i, _RVECS - 1), 4), _L)
                sl = pl.ds(col, _L)
                m = mb[r, sl]
                lv = lvb[r, sl]
                t = tb[r, sl]
                msk = t > 0.0
                d = t - m
                e = jnp.exp(-lv) * (d * d) + lv
                return (s + jnp.where(msk, e, 0.0),
                        n + jnp.where(msk, 1.0, 0.0))

            return lax.fori_loop(0, _VECS, body, carry, unroll=4)

        zero = jnp.zeros((_L,), jnp.float32)
        start(0, 0)

        def pair_body(j, carry):
            c0 = 2 * j
            start(c0 + 1, 1)
            wait(c0, 0)
            carry = compute(0, carry)

            @pl.when(c0 + 2 < _NCHUNK)
            def _():
                start(c0 + 2, 0)

            wait(c0 + 1, 1)
            return compute(1, carry)

        s, n = lax.fori_loop(0, _NCHUNK // 2, pair_body, (zero, zero))
        acc[0, :] = s
        acc[1, :] = n
        pltpu.sync_copy(acc, out_hbm.at[wid])

    return masked_gauss


_sc_call = _make_sc_kernel()


_RING = 5
_RG = 32
_TC_BLOCKS = [(b, r0)
              for b in range(_B)
              for r0 in range(_R_SC, _H, _BR)]


def _tc_body(m_hbm, lv_hbm, t_hbm, out_ref, mb, lvb, tb, sems):
    nb = len(_TC_BLOCKS)

    def start(g):
        b, r0 = _TC_BLOCKS[g]
        slot = g % _RING
        rows = pl.ds(r0, _BR)
        return (
            pltpu.async_copy(m_hbm.at[b, 0, rows], mb.at[slot], sems.at[slot]),
            pltpu.async_copy(lv_hbm.at[b, 0, rows], lvb.at[slot],
                             sems.at[slot]),
            pltpu.async_copy(t_hbm.at[b, rows], tb.at[slot], sems.at[slot]),
        )

    ahead = _RING - 1
    handles = [None] * nb
    for g in range(min(ahead, nb)):
        handles[g] = start(g)
    s = jnp.zeros((8, _W), jnp.float32)
    c = jnp.zeros((8, _W), jnp.float32)
    for g in range(nb):
        if g + ahead < nb:
            handles[g + ahead] = start(g + ahead)
        for h in handles[g]:
            h.wait()
        handles[g] = None
        slot = g % _RING
        for r in range(0, _BR, _RG):
            rows = pl.ds(r, _RG)
            m = mb[slot, rows, :]
            lv = lvb[slot, rows, :]
            t = tb[slot, rows, :]
            msk = t > 0.0
            d = t - m
            e = jnp.exp(-lv) * (d * d) + lv
            s = s + jnp.where(msk, e, 0.0).reshape(_RG // 8, 8, _W).sum(0)
            c = c + jnp.where(msk, 1.0, 0.0).reshape(_RG // 8, 8, _W).sum(0)
    out_ref[0, 0] = jnp.sum(s) / jnp.sum(c)


_tc_call = pl.pallas_call(
    _tc_body,
    in_specs=[
        pl.BlockSpec(memory_space=pl.ANY),
        pl.BlockSpec(memory_space=pl.ANY),
        pl.BlockSpec(memory_space=pl.ANY),
    ],
    out_specs=pl.BlockSpec(memory_space=pltpu.MemorySpace.SMEM),
    out_shape=jax.ShapeDtypeStruct((1, 1), jnp.float32),
    scratch_shapes=[
        pltpu.VMEM((_RING, _BR, _W), jnp.float32),
        pltpu.VMEM((_RING, _BR, _W), jnp.float32),
        pltpu.VMEM((_RING, _BR, _W), jnp.float32),
        pltpu.SemaphoreType.DMA((_RING,)),
    ],
)


@jax.jit
def kernel(means, log_vars, targets):
    loss = _tc_call(means, log_vars, targets)
    return loss[0, 0]
